# baseline (speedup 1.0000x reference)
import functools

import jax
import jax.numpy as jnp
from jax import lax
from jax.experimental import pallas as pl
from jax.experimental.pallas import tpu as pltpu
from jax.experimental.pallas import tpu_sc as plsc

HIDDEN = 512
NUM_WORKERS = 32
CHUNK = 64
GROUP = 8
STREAMED_PER_GROUP = 2
TABLE_REPLICAS = 64
ROW_UNROLL = 4


def _emb_body(
    idx_raw_hbm,
    idx_spread_hbm,
    table_rep_hbm,
    table_hbm,
    out_hbm,
    tab_v,
    i0,
    i1,
    r0,
    r1,
    g0,
    g1,
    w0,
    w1,
):
    wid = lax.axis_index("s") * 2 + lax.axis_index("c")
    per_w = idx_raw_hbm.shape[0] // NUM_WORKERS
    base = wid * per_w
    nchunk = per_w // CHUNK
    ngroup = nchunk // GROUP
    idxb = (i0, i1)
    rows = (r0, r1)
    gsem = (g0, g1)
    wsem = (w0, w1)

    pltpu.sync_copy(table_hbm, tab_v)

    iota16 = lax.broadcasted_iota(jnp.int32, (16,), 0)

    def wait_write(b):
        pltpu.make_async_copy(rows[b], out_hbm.at[pl.ds(0, CHUNK)], wsem[b]).wait()

    def tec_fill(b):
        @plsc.parallel_loop(0, CHUNK, 1, unroll=ROW_UNROLL)
        def row_body(r):
            s = idxb[b][pl.ds(r, 16)][0]
            sbase = s * HIDDEN
            for k in range(HIDDEN // 16):
                rows[b][r, pl.ds(16 * k, 16)] = tab_v[pl.ds(sbase + 16 * k, 16)]

    def do_chunk(i, b, streamed, first):
        off = base + i * CHUNK
        if not first:
            wait_write(b)
        if streamed:
            pltpu.sync_copy(idx_spread_hbm.at[pl.ds(off, CHUNK)], idxb[b].at[pl.ds(0, CHUNK)])
            pltpu.async_copy(
                table_rep_hbm.at[idxb[b].at[pl.ds(0, CHUNK)]], rows[b], gsem[b]
            ).wait()
        else:
            pltpu.sync_copy(idx_raw_hbm.at[pl.ds(off, CHUNK)], idxb[b].at[pl.ds(0, CHUNK)])
            tec_fill(b)
        pltpu.async_copy(rows[b], out_hbm.at[pl.ds(off, CHUNK)], wsem[b])

    def run_group(g, first):
        for j in range(GROUP):
            streamed = j % (GROUP // STREAMED_PER_GROUP) == 0
            do_chunk(g * GROUP + j, j % 2, streamed, first and j < 2)

    run_group(0, True)

    def group_body(g, carry):
        run_group(g, False)
        return carry

    lax.fori_loop(1, ngroup, group_body, 0)

    for b in range(2):
        wait_write(b)


def _make_emb(n_idx):
    return functools.partial(
        pl.kernel,
        mesh=plsc.VectorSubcoreMesh(core_axis_name="c", subcore_axis_name="s"),
        out_type=jax.ShapeDtypeStruct((n_idx, HIDDEN), jnp.float32),
        scratch_types=[
            pltpu.VMEM((101 * HIDDEN,), jnp.float32),
            pltpu.VMEM((128,), jnp.int32),
            pltpu.VMEM((128,), jnp.int32),
            pltpu.VMEM((CHUNK, HIDDEN), jnp.float32),
            pltpu.VMEM((CHUNK, HIDDEN), jnp.float32),
            pltpu.SemaphoreType.DMA,
            pltpu.SemaphoreType.DMA,
            pltpu.SemaphoreType.DMA,
            pltpu.SemaphoreType.DMA,
        ],
    )(_emb_body)


def kernel(atom_types, embedding_table):
    b, n = atom_types.shape
    idx = atom_types.reshape(-1).astype(jnp.int32)
    nrows = embedding_table.shape[0]
    table_rep = jnp.tile(embedding_table, (TABLE_REPLICAS, 1))
    spread = (jnp.arange(idx.shape[0], dtype=jnp.int32) % TABLE_REPLICAS) * nrows
    out = _make_emb(idx.shape[0])(
        idx, idx + spread, table_rep, embedding_table.reshape(-1)
    )
    return out.reshape(b, n, HIDDEN)


# speedup vs baseline: 1.0342x; 1.0342x over previous
import functools

import jax
import jax.numpy as jnp
from jax import lax
from jax.experimental import pallas as pl
from jax.experimental.pallas import tpu as pltpu
from jax.experimental.pallas import tpu_sc as plsc

HIDDEN = 512
NUM_WORKERS = 32
CHUNK = 64
GROUP = 8
STREAM_PATTERN = (1, 0, 1, 0, 0, 1, 0, 0)
TABLE_REPLICAS = 64
ROW_UNROLL = 2


def _emb_body(
    idx_raw_hbm,
    idx_spread_hbm,
    table_rep_hbm,
    table_hbm,
    out_hbm,
    tab_v,
    i0,
    i1,
    r0,
    r1,
    g0,
    g1,
    w0,
    w1,
):
    wid = lax.axis_index("s") * 2 + lax.axis_index("c")
    per_w = idx_raw_hbm.shape[0] // NUM_WORKERS
    base = wid * per_w
    nchunk = per_w // CHUNK
    ngroup = nchunk // GROUP
    idxb = (i0, i1)
    rows = (r0, r1)
    gsem = (g0, g1)
    wsem = (w0, w1)

    pltpu.sync_copy(table_hbm, tab_v)

    iota16 = lax.broadcasted_iota(jnp.int32, (16,), 0)

    def wait_write(b):
        pltpu.make_async_copy(rows[b], out_hbm.at[pl.ds(0, CHUNK)], wsem[b]).wait()

    def tec_fill(b):
        @plsc.parallel_loop(0, CHUNK, 1, unroll=ROW_UNROLL)
        def row_body(r):
            s = idxb[b][pl.ds(r, 16)][0]
            sbase = s * HIDDEN
            for k in range(HIDDEN // 16):
                rows[b][r, pl.ds(16 * k, 16)] = tab_v[pl.ds(sbase + 16 * k, 16)]

    def do_chunk(i, b, streamed, first):
        off = base + i * CHUNK
        if not first:
            wait_write(b)
        if streamed:
            pltpu.sync_copy(idx_spread_hbm.at[pl.ds(off, CHUNK)], idxb[b].at[pl.ds(0, CHUNK)])
            pltpu.async_copy(
                table_rep_hbm.at[idxb[b].at[pl.ds(0, CHUNK)]], rows[b], gsem[b]
            ).wait()
        else:
            pltpu.sync_copy(idx_raw_hbm.at[pl.ds(off, CHUNK)], idxb[b].at[pl.ds(0, CHUNK)])
            tec_fill(b)
        pltpu.async_copy(rows[b], out_hbm.at[pl.ds(off, CHUNK)], wsem[b])

    def run_group(g, first):
        for j in range(GROUP):
            streamed = bool(STREAM_PATTERN[j])
            do_chunk(g * GROUP + j, j % 2, streamed, first and j < 2)

    run_group(0, True)

    def group_body(g, carry):
        run_group(g, False)
        return carry

    lax.fori_loop(1, ngroup, group_body, 0)

    for b in range(2):
        wait_write(b)


def _make_emb(n_idx):
    return functools.partial(
        pl.kernel,
        mesh=plsc.VectorSubcoreMesh(core_axis_name="c", subcore_axis_name="s"),
        out_type=jax.ShapeDtypeStruct((n_idx, HIDDEN), jnp.float32),
        scratch_types=[
            pltpu.VMEM((101 * HIDDEN,), jnp.float32),
            pltpu.VMEM((128,), jnp.int32),
            pltpu.VMEM((128,), jnp.int32),
            pltpu.VMEM((CHUNK, HIDDEN), jnp.float32),
            pltpu.VMEM((CHUNK, HIDDEN), jnp.float32),
            pltpu.SemaphoreType.DMA,
            pltpu.SemaphoreType.DMA,
            pltpu.SemaphoreType.DMA,
            pltpu.SemaphoreType.DMA,
        ],
    )(_emb_body)


def kernel(atom_types, embedding_table):
    b, n = atom_types.shape
    idx = atom_types.reshape(-1).astype(jnp.int32)
    nrows = embedding_table.shape[0]
    table_rep = jnp.tile(embedding_table, (TABLE_REPLICAS, 1))
    spread = (jnp.arange(idx.shape[0], dtype=jnp.int32) % TABLE_REPLICAS) * nrows
    out = _make_emb(idx.shape[0])(
        idx, idx + spread, table_rep, embedding_table.reshape(-1)
    )
    return out.reshape(b, n, HIDDEN)


# speedup vs baseline: 1.1184x; 1.0814x over previous
import functools

import jax
import jax.numpy as jnp
from jax import lax
from jax.experimental import pallas as pl
from jax.experimental.pallas import tpu as pltpu
from jax.experimental.pallas import tpu_sc as plsc

HIDDEN = 512
NUM_WORKERS = 32
CHUNK = 40
TABLE_REPLICAS = 64
ROW_UNROLL = 2
WBYTES = CHUNK * HIDDEN * 4


def _emb_body(
    idx_raw_hbm,
    idx_spread_hbm,
    table_rep_hbm,
    table_hbm,
    out_hbm,
    tab_v,
    idxg,
    idxt,
    gbuf,
    t0,
    t1,
    gsem,
    wg,
    w0,
    w1,
):
    wid = lax.axis_index("s") * 2 + lax.axis_index("c")
    per_w = idx_raw_hbm.shape[0] // NUM_WORKERS
    base = wid * per_w
    nchunk = per_w // CHUNK
    ngroup = nchunk // 3

    pltpu.sync_copy(table_hbm, tab_v)

    def wait_write(buf, w):
        pltpu.make_async_copy(buf, out_hbm.at[pl.ds(0, CHUNK)], w).wait()

    def tec_fill(buf):
        @plsc.parallel_loop(0, CHUNK, 1, unroll=ROW_UNROLL)
        def row_body(r):
            s = idxt[pl.ds(r, 16)][0]
            sbase = s * HIDDEN
            for k in range(HIDDEN // 16):
                buf[r, pl.ds(16 * k, 16)] = tab_v[pl.ds(sbase + 16 * k, 16)]

    def tec_chunk(i, buf, w, first=False):
        off = base + i * CHUNK
        if not first:
            wait_write(buf, w)
        pltpu.sync_copy(idx_raw_hbm.at[pl.ds(off, CHUNK)], idxt.at[pl.ds(0, CHUNK)])
        tec_fill(buf)
        pltpu.async_copy(buf, out_hbm.at[pl.ds(off, CHUNK)], w)

    def run_group(g, first=False):
        c0 = g * 3
        if not first:
            wait_write(gbuf, wg)
        pltpu.sync_copy(
            idx_spread_hbm.at[pl.ds(base + c0 * CHUNK, CHUNK)],
            idxg.at[pl.ds(0, CHUNK)],
        )
        pltpu.async_copy(table_rep_hbm.at[idxg.at[pl.ds(0, CHUNK)]], gbuf, gsem)
        tec_chunk(c0 + 1, t0, w0, first)
        pltpu.make_async_copy(
            table_rep_hbm.at[idxg.at[pl.ds(0, CHUNK)]], gbuf, gsem
        ).wait()
        pltpu.async_copy(gbuf, out_hbm.at[pl.ds(base + c0 * CHUNK, CHUNK)], wg)
        tec_chunk(c0 + 2, t1, w1, first)

    run_group(0, first=True)

    def group_body(g, carry):
        run_group(g)
        return carry

    lax.fori_loop(1, ngroup, group_body, 0)

    for i in range(ngroup * 3, nchunk):
        tec_chunk(i, t0, w0)

    wait_write(gbuf, wg)
    wait_write(t0, w0)
    wait_write(t1, w1)


def _make_emb(n_idx):
    return functools.partial(
        pl.kernel,
        mesh=plsc.VectorSubcoreMesh(core_axis_name="c", subcore_axis_name="s"),
        out_type=jax.ShapeDtypeStruct((n_idx, HIDDEN), jnp.float32),
        scratch_types=[
            pltpu.VMEM((101 * HIDDEN,), jnp.float32),
            pltpu.VMEM((128,), jnp.int32),
            pltpu.VMEM((128,), jnp.int32),
            pltpu.VMEM((CHUNK, HIDDEN), jnp.float32),
            pltpu.VMEM((CHUNK, HIDDEN), jnp.float32),
            pltpu.VMEM((CHUNK, HIDDEN), jnp.float32),
            pltpu.SemaphoreType.DMA,
            pltpu.SemaphoreType.DMA,
            pltpu.SemaphoreType.DMA,
            pltpu.SemaphoreType.DMA,
        ],
    )(_emb_body)


def kernel(atom_types, embedding_table):
    b, n = atom_types.shape
    idx = atom_types.reshape(-1).astype(jnp.int32)
    nrows = embedding_table.shape[0]
    table_rep = jnp.tile(embedding_table, (TABLE_REPLICAS, 1))
    spread = (jnp.arange(idx.shape[0], dtype=jnp.int32) % TABLE_REPLICAS) * nrows
    out = _make_emb(idx.shape[0])(
        idx, idx + spread, table_rep, embedding_table.reshape(-1)
    )
    return out.reshape(b, n, HIDDEN)


# speedup vs baseline: 1.1338x; 1.0137x over previous
import functools

import jax
import jax.numpy as jnp
from jax import lax
from jax.experimental import pallas as pl
from jax.experimental.pallas import tpu as pltpu
from jax.experimental.pallas import tpu_sc as plsc

HIDDEN = 512
NUM_WORKERS = 32
CHUNK = 32
TABLE_REPLICAS = 64
ROW_UNROLL = 2
NROWS = 101


def _emb_body(
    idx_hbm,
    table_rep_hbm,
    table_hbm,
    out_hbm,
    tab_v,
    idx_v,
    idxg,
    gbuf,
    t0,
    t1,
    gsem,
    wg,
    w0,
    w1,
):
    wid = lax.axis_index("s") * 2 + lax.axis_index("c")
    per_w = idx_hbm.shape[0] // NUM_WORKERS
    base = wid * per_w
    nchunk = per_w // CHUNK
    ngroup = nchunk // 3

    pltpu.sync_copy(table_hbm, tab_v)
    pltpu.sync_copy(idx_hbm.at[pl.ds(base, per_w)], idx_v.at[pl.ds(0, per_w)])

    iota16 = lax.broadcasted_iota(jnp.int32, (16,), 0)

    def wait_write(buf, w):
        pltpu.make_async_copy(buf, out_hbm.at[pl.ds(0, CHUNK)], w).wait()

    def tec_fill(i, buf):
        loc = i * CHUNK

        @plsc.parallel_loop(0, CHUNK, 1, unroll=ROW_UNROLL)
        def row_body(r):
            s = idx_v[pl.ds(loc + r, 16)][0]
            sbase = s * HIDDEN
            for k in range(HIDDEN // 16):
                buf[r, pl.ds(16 * k, 16)] = tab_v[pl.ds(sbase + 16 * k, 16)]

    def tec_chunk(i, buf, w, first=False):
        if not first:
            wait_write(buf, w)
        tec_fill(i, buf)
        pltpu.async_copy(buf, out_hbm.at[pl.ds(base + i * CHUNK, CHUNK)], w)

    def run_group(g, first=False):
        c0 = g * 3
        if not first:
            wait_write(gbuf, wg)
        loc = c0 * CHUNK
        for q in range(CHUNK // 16):
            rep = (base + loc + 16 * q + iota16) & (TABLE_REPLICAS - 1)
            idxg[pl.ds(16 * q, 16)] = idx_v[pl.ds(loc + 16 * q, 16)] + rep * NROWS
        pltpu.async_copy(table_rep_hbm.at[idxg.at[pl.ds(0, CHUNK)]], gbuf, gsem)
        tec_chunk(c0 + 1, t0, w0, first)
        pltpu.make_async_copy(
            table_rep_hbm.at[idxg.at[pl.ds(0, CHUNK)]], gbuf, gsem
        ).wait()
        pltpu.async_copy(gbuf, out_hbm.at[pl.ds(base + loc, CHUNK)], wg)
        tec_chunk(c0 + 2, t1, w1, first)

    run_group(0, first=True)

    def group_body(g, carry):
        run_group(g)
        return carry

    lax.fori_loop(1, ngroup, group_body, 0)

    for i in range(ngroup * 3, nchunk):
        tec_chunk(i, t0, w0)

    wait_write(gbuf, wg)
    wait_write(t0, w0)
    wait_write(t1, w1)


def _make_emb(n_idx):
    per_w = n_idx // NUM_WORKERS
    return functools.partial(
        pl.kernel,
        mesh=plsc.VectorSubcoreMesh(core_axis_name="c", subcore_axis_name="s"),
        out_type=jax.ShapeDtypeStruct((n_idx, HIDDEN), jnp.float32),
        scratch_types=[
            pltpu.VMEM((NROWS * HIDDEN,), jnp.float32),
            pltpu.VMEM((per_w + 16,), jnp.int32),
            pltpu.VMEM((128,), jnp.int32),
            pltpu.VMEM((CHUNK, HIDDEN), jnp.float32),
            pltpu.VMEM((CHUNK, HIDDEN), jnp.float32),
            pltpu.VMEM((CHUNK, HIDDEN), jnp.float32),
            pltpu.SemaphoreType.DMA,
            pltpu.SemaphoreType.DMA,
            pltpu.SemaphoreType.DMA,
            pltpu.SemaphoreType.DMA,
        ],
    )(_emb_body)


def kernel(atom_types, embedding_table):
    b, n = atom_types.shape
    idx = atom_types.reshape(-1).astype(jnp.int32)
    table_rep = jnp.tile(embedding_table, (TABLE_REPLICAS, 1))
    out = _make_emb(idx.shape[0])(idx, table_rep, embedding_table.reshape(-1))
    return out.reshape(b, n, HIDDEN)
